# baseline (speedup 1.0000x reference)
import functools

import jax
import jax.numpy as jnp
from jax import lax
from jax.experimental import pallas as pl
from jax.experimental.pallas import tpu as pltpu
from jax.experimental.pallas import tpu_sc as plsc

SLICE = 64
L = 16


def _sc_slice_gather(n, d, rows_per_w, chunk_rows, nc):
    n_chunks = rows_per_w // chunk_rows
    assert n_chunks % 2 == 0
    groups = chunk_rows // L
    j_steps = SLICE // L

    mesh = plsc.VectorSubcoreMesh(core_axis_name="c", subcore_axis_name="s")

    @functools.partial(
        pl.kernel,
        mesh=mesh,
        compiler_params=pltpu.CompilerParams(needs_layout_passes=False),
        out_type=jax.ShapeDtypeStruct((n // 2, 2 * SLICE), jnp.float32),
        scratch_types=[
            pltpu.VMEM((chunk_rows, d), jnp.float32),
            pltpu.VMEM((chunk_rows, d), jnp.float32),
            pltpu.VMEM((chunk_rows // 2, 2 * SLICE), jnp.float32),
            pltpu.VMEM((chunk_rows // 2, 2 * SLICE), jnp.float32),
            pltpu.VMEM((chunk_rows,), jnp.int32),
            pltpu.VMEM((chunk_rows,), jnp.int32),
            pltpu.SemaphoreType.DMA,
            pltpu.SemaphoreType.DMA,
            pltpu.SemaphoreType.DMA,
            pltpu.SemaphoreType.DMA,
        ],
    )
    def k(in_hbm, idx_hbm, out_hbm, in_v0, in_v1, out_v0, out_v1,
          idx_v0, idx_v1, sem_in0, sem_in1, sem_out0, sem_out1):
        in_v = (in_v0, in_v1)
        out_v = (out_v0, out_v1)
        idx_v = (idx_v0, idx_v1)
        sem_in = (sem_in0, sem_in1)
        sem_out = (sem_out0, sem_out1)
        wid = lax.axis_index("s") * nc + lax.axis_index("c")
        base_row = wid * rows_per_w
        iota = lax.iota(jnp.int32, L)

        def in_copy(c, b):
            row0 = base_row + c * chunk_rows
            return (
                pltpu.make_async_copy(
                    in_hbm.at[pl.ds(row0, chunk_rows)], in_v[b], sem_in[b]),
                pltpu.make_async_copy(
                    idx_hbm.at[pl.ds(row0, chunk_rows)], idx_v[b], sem_in[b]),
            )

        def out_copy(c, b):
            row0 = base_row + c * chunk_rows
            return pltpu.make_async_copy(
                out_v[b],
                out_hbm.at[pl.ds(pl.multiple_of(row0 // 2, 8),
                                 chunk_rows // 2)],
                sem_out[b])

        def compute(b):
            @plsc.parallel_loop(0, groups, 1)
            def group_body(g):
                svec = idx_v[b][pl.ds(g * L, L)]
                for r in range(L):
                    row = g * L + r
                    s_b = jnp.take_along_axis(
                        svec, jnp.full((L,), r, jnp.int32), axis=0)
                    rvec = jnp.full((L,), row, jnp.int32)
                    col0 = s_b + iota
                    half = (r % 2) * SLICE
                    for j in range(j_steps):
                        vals = plsc.load_gather(
                            in_v[b], [rvec, col0 + (j * L)])
                        out_v[b][g * (L // 2) + r // 2,
                                 pl.ds(half + j * L, L)] = vals

        for b in range(2):
            for cp in in_copy(b, b):
                cp.start()

        def pair_body(i, carry):
            for b in range(2):
                c = i * 2 + b
                for cp in in_copy(c, b):
                    cp.wait()

                @pl.when(i > 0)
                def _():
                    out_copy(c, b).wait()

                compute(b)
                out_copy(c, b).start()

                @pl.when(c + 2 < n_chunks)
                def _():
                    for cp in in_copy(c + 2, b):
                        cp.start()
            return carry

        lax.fori_loop(0, n_chunks // 2, pair_body, 0)
        for b in range(2):
            out_copy(n_chunks - 2 + b, b).wait()

    return k


def _tc_unpack(n, blk):
    def body(in_ref, out_ref):
        x = in_ref[...]
        y = jnp.stack([x[:, :SLICE], x[:, SLICE:]], axis=1)
        out_ref[...] = y.reshape(blk, SLICE)

    return pl.pallas_call(
        body,
        grid=(n // blk,),
        in_specs=[pl.BlockSpec((blk // 2, 2 * SLICE), lambda i: (i, 0))],
        out_specs=pl.BlockSpec((blk, SLICE), lambda i: (i, 0)),
        out_shape=jax.ShapeDtypeStruct((n, SLICE), jnp.float32),
    )


def kernel(input_tensor, slices_index, slice_len):
    n, d = input_tensor.shape
    adj_idx = slices_index.astype(jnp.int32) + (
        jnp.asarray(slice_len, jnp.int32) - SLICE)

    num_workers = 32
    nc = 2
    rows_per_w = n // num_workers
    chunk_rows = 128
    f = _sc_slice_gather(n, d, rows_per_w, chunk_rows, nc)
    packed = f(input_tensor, adj_idx)
    return _tc_unpack(n, 2048)(packed)


# speedup vs baseline: 1.7582x; 1.7582x over previous
import functools

import jax
import jax.numpy as jnp
from jax import lax
from jax.experimental import pallas as pl
from jax.experimental.pallas import tpu as pltpu
from jax.experimental.pallas import tpu_sc as plsc

SLICE = 64
L = 16


def _sc_slice_gather(n, d, rows_per_w, chunk_rows, nc):
    n_chunks = rows_per_w // chunk_rows
    assert n_chunks % 4 == 0
    groups = chunk_rows // L
    j_steps = SLICE // L

    mesh = plsc.VectorSubcoreMesh(core_axis_name="c", subcore_axis_name="s")

    @functools.partial(
        pl.kernel,
        mesh=mesh,
        compiler_params=pltpu.CompilerParams(needs_layout_passes=False),
        out_type=jax.ShapeDtypeStruct((n, SLICE), jnp.float32),
        scratch_types=[
            pltpu.VMEM((chunk_rows, d), jnp.float32),
            pltpu.VMEM((chunk_rows, d), jnp.float32),
            pltpu.VMEM((chunk_rows, d), jnp.float32),
            pltpu.VMEM((chunk_rows, d), jnp.float32),
            pltpu.VMEM((chunk_rows, SLICE), jnp.float32),
            pltpu.VMEM((chunk_rows, SLICE), jnp.float32),
            pltpu.VMEM((chunk_rows,), jnp.int32),
            pltpu.VMEM((chunk_rows,), jnp.int32),
            pltpu.VMEM((chunk_rows,), jnp.int32),
            pltpu.VMEM((chunk_rows,), jnp.int32),
            pltpu.SemaphoreType.DMA,
            pltpu.SemaphoreType.DMA,
            pltpu.SemaphoreType.DMA,
            pltpu.SemaphoreType.DMA,
            pltpu.SemaphoreType.DMA,
            pltpu.SemaphoreType.DMA,
        ],
    )
    def k(in_hbm, idx_hbm, out_hbm, in_v0, in_v1, in_v2, in_v3,
          out_v0, out_v1, idx_v0, idx_v1, idx_v2, idx_v3,
          sem_in0, sem_in1, sem_in2, sem_in3, sem_out0, sem_out1):
        in_v = (in_v0, in_v1, in_v2, in_v3)
        out_v = (out_v0, out_v1)
        idx_v = (idx_v0, idx_v1, idx_v2, idx_v3)
        sem_in = (sem_in0, sem_in1, sem_in2, sem_in3)
        sem_out = (sem_out0, sem_out1)
        wid = lax.axis_index("s") * nc + lax.axis_index("c")
        base_row = wid * rows_per_w
        iota = lax.iota(jnp.int32, L)

        def in_copy(c, b):
            row0 = base_row + c * chunk_rows
            return (
                pltpu.make_async_copy(
                    in_hbm.at[pl.ds(row0, chunk_rows)], in_v[b], sem_in[b]),
                pltpu.make_async_copy(
                    idx_hbm.at[pl.ds(row0, chunk_rows)], idx_v[b], sem_in[b]),
            )

        def out_copy(c, b):
            row0 = base_row + c * chunk_rows
            return pltpu.make_async_copy(
                out_v[b], out_hbm.at[pl.ds(row0, chunk_rows)], sem_out[b])

        def compute(b, bo):
            @plsc.parallel_loop(0, groups, 1)
            def group_body(g):
                svec = idx_v[b][pl.ds(g * L, L)]
                for r in range(L):
                    row = g * L + r
                    s_b = jnp.take_along_axis(
                        svec, jnp.full((L,), r, jnp.int32), axis=0)
                    rvec = jnp.full((L,), row, jnp.int32)
                    col0 = s_b + iota
                    for j in range(j_steps):
                        vals = plsc.load_gather(
                            in_v[b], [rvec, col0 + (j * L)])
                        out_v[bo][row, pl.ds(j * L, L)] = vals

        for b in range(4):
            for cp in in_copy(b, b):
                cp.start()

        def quad_body(i, carry):
            for cc in range(4):
                c = i * 4 + cc
                bo = cc % 2
                for cp in in_copy(c, cc):
                    cp.wait()

                @pl.when(c >= 2)
                def _():
                    out_copy(c, bo).wait()

                compute(cc, bo)
                out_copy(c, bo).start()

                @pl.when(c + 4 < n_chunks)
                def _():
                    for cp in in_copy(c + 4, cc):
                        cp.start()
            return carry

        lax.fori_loop(0, n_chunks // 4, quad_body, 0)
        for bo in range(2):
            out_copy(n_chunks - 2 + bo, bo).wait()

    return k


def kernel(input_tensor, slices_index, slice_len):
    n, d = input_tensor.shape
    adj_idx = slices_index.astype(jnp.int32) + (
        jnp.asarray(slice_len, jnp.int32) - SLICE)

    num_workers = 32
    nc = 2
    rows_per_w = n // num_workers
    chunk_rows = 128
    f = _sc_slice_gather(n, d, rows_per_w, chunk_rows, nc)
    return f(input_tensor, adj_idx)
